# baseline (speedup 1.0000x reference)
import functools

import jax
import jax.numpy as jnp
from jax import lax
from jax.experimental import pallas as pl
from jax.experimental.pallas import tpu as pltpu
from jax.experimental.pallas import tpu_sc as plsc

_L = 16
_NW = 32
_C = 128


def kernel(x, weights, table):
    B, F = x.shape
    V, D = table.shape
    n_total = B * F
    per_w = n_total // _NW
    n_chunks = per_w // _C
    assert n_total % _NW == 0 and per_w % _C == 0 and D % _L == 0

    x_split = x.reshape(_NW, n_chunks, _C)
    w_pad = jnp.zeros((_L,), jnp.float32).at[: weights.shape[0]].set(weights)

    mesh = plsc.VectorSubcoreMesh(core_axis_name="c", subcore_axis_name="s")

    @functools.partial(
        pl.kernel,
        mesh=mesh,
        out_type=jax.ShapeDtypeStruct((n_total, D), jnp.float32),
        scratch_types=[
            pltpu.VMEM((n_chunks, _C), jnp.int32),
            pltpu.VMEM((_L,), jnp.float32),
            pltpu.VMEM((2, _C, D), jnp.float32),
            pltpu.SemaphoreType.DMA,
            pltpu.SemaphoreType.DMA,
            pltpu.SemaphoreType.DMA,
            pltpu.SemaphoreType.DMA,
        ],
    )
    def run(x_hbm, w_hbm, table_hbm, out_hbm, idx_v, w_v, rows_v, g0, g1, p0, p1):
        wid = lax.axis_index("s") * 2 + lax.axis_index("c")
        base = wid * per_w

        pltpu.sync_copy(w_hbm, w_v)
        pltpu.sync_copy(x_hbm.at[wid], idx_v)

        ones = jnp.ones((_L,), jnp.float32)
        w_vec = w_v[...]
        w0, w1, w2 = w_vec[0] * ones, w_vec[1] * ones, w_vec[2] * ones
        s2 = w2
        s1 = w1 + s2
        s0 = w0 + s1
        scales = [s0, s0, s1, s1, s2, s2, s2, s2]

        gsem = [g0, g1]
        wsem = [p0, p1]

        def gather(j):
            b = j % 2
            return pltpu.async_copy(table_hbm.at[idx_v.at[j]], rows_v.at[b], gsem[b])

        def write(j):
            b = j % 2
            return pltpu.async_copy(
                rows_v.at[b], out_hbm.at[pl.ds(base + j * _C, _C)], wsem[b]
            )

        def scale_buf(b):
            @plsc.parallel_loop(0, _C, 1, unroll=4)
            def _(i):
                for jc in range(D // _L):
                    sl = pl.ds(jc * _L, _L)
                    rows_v[b, i, sl] = rows_v[b, i, sl] * scales[jc]

        pend_w = [None, None]
        gh = [None, None]
        gh[0] = gather(0)
        for j in range(n_chunks):
            b = j % 2
            nb = (j + 1) % 2
            if j + 1 < n_chunks:
                if pend_w[nb] is not None:
                    pend_w[nb].wait()
                    pend_w[nb] = None
                gh[nb] = gather(j + 1)
            gh[b].wait()
            scale_buf(b)
            pend_w[b] = write(j)
        for b in range(2):
            if pend_w[b] is not None:
                pend_w[b].wait()

    out_flat = run(x_split, w_pad, table)
    return out_flat.reshape(B, F, D)


# speedup vs baseline: 1.4731x; 1.4731x over previous
import functools

import jax
import jax.numpy as jnp
from jax import lax
from jax.experimental import pallas as pl
from jax.experimental.pallas import tpu as pltpu
from jax.experimental.pallas import tpu_sc as plsc

_L = 16
_NW = 32
_CB = 4


def kernel(x, weights, table):
    B, F = x.shape
    V, D = table.shape
    rows_per_chunk = _CB * F
    batches_per_w = B // _NW
    n_chunks = batches_per_w // _CB
    assert B % (_NW * _CB) == 0 and rows_per_chunk % 8 == 0 and D % _L == 0

    x_split = x.reshape(_NW, n_chunks, 1, _CB * F)
    w_pad = jnp.zeros((_L,), jnp.float32).at[: weights.shape[0]].set(weights)

    mesh = plsc.VectorSubcoreMesh(core_axis_name="c", subcore_axis_name="s")

    @functools.partial(
        pl.kernel,
        mesh=mesh,
        out_type=jax.ShapeDtypeStruct((B, F, D), jnp.float32),
        scratch_types=[
            pltpu.VMEM((n_chunks, 1, rows_per_chunk), jnp.int32),
            pltpu.VMEM((_L,), jnp.float32),
            pltpu.VMEM((2, _CB * F, D), jnp.float32),
            pltpu.SemaphoreType.DMA,
            pltpu.SemaphoreType.DMA,
            pltpu.SemaphoreType.DMA,
            pltpu.SemaphoreType.DMA,
        ],
    )
    def run(x_hbm, w_hbm, table_hbm, out_hbm, idx_v, w_v, rows_v, g0, g1, p0, p1):
        wid = lax.axis_index("s") * 2 + lax.axis_index("c")
        batch0 = wid * batches_per_w

        pltpu.sync_copy(w_hbm, w_v)
        pltpu.sync_copy(x_hbm.at[wid], idx_v)

        ones = jnp.ones((_L,), jnp.float32)
        w_vec = w_v[...]
        w0, w1, w2 = w_vec[0] * ones, w_vec[1] * ones, w_vec[2] * ones
        s2 = w2
        s1 = w1 + s2
        s0 = w0 + s1
        scales = [s0, s0, s1, s1, s2, s2, s2, s2]

        gsem = [g0, g1]
        wsem = [p0, p1]
        def gather(j):
            b = j % 2
            return pltpu.async_copy(
                table_hbm.at[idx_v.at[j, 0]], rows_v.at[b], gsem[b]
            )

        def write(j):
            b = j % 2
            last = None
            for bb in range(_CB):
                last = pltpu.async_copy(
                    rows_v.at[b, pl.ds(bb * F, F)],
                    out_hbm.at[batch0 + j * _CB + bb],
                    wsem[b],
                )
            return last

        def scale_buf(b):
            @plsc.parallel_loop(0, _CB * F, 1, unroll=4)
            def _(i):
                for jc in range(D // _L):
                    sl = pl.ds(jc * _L, _L)
                    rows_v[b, i, sl] = rows_v[b, i, sl] * scales[jc]

        pend_w = [None, None]
        gh = [None, None]
        gh[0] = gather(0)
        for j in range(n_chunks):
            b = j % 2
            nb = (j + 1) % 2
            if j + 1 < n_chunks:
                if pend_w[nb] is not None:
                    for h in pend_w[nb]:
                        h.wait()
                    pend_w[nb] = None
                gh[nb] = gather(j + 1)
            gh[b].wait()
            scale_buf(b)
            write(j)
            pend_w[b] = [
                pltpu.make_async_copy(
                    rows_v.at[b, pl.ds(bb * F, F)],
                    out_hbm.at[batch0 + j * _CB + bb],
                    wsem[b],
                )
                for bb in range(_CB)
            ]
        for b in range(2):
            if pend_w[b] is not None:
                for h in pend_w[b]:
                    h.wait()

    return run(x_split, w_pad, table)


# speedup vs baseline: 2.5947x; 1.7614x over previous
import functools

import jax
import jax.numpy as jnp
from jax import lax
from jax.experimental import pallas as pl
from jax.experimental.pallas import tpu as pltpu
from jax.experimental.pallas import tpu_sc as plsc

_L = 16
_NW = 32
_C = 128


def kernel(x, weights, table):
    B, F = x.shape
    V, D = table.shape
    n_total = B * F
    per_w = n_total // _NW
    n_chunks = per_w // _C
    chunks_per_f = B // _C
    assert n_total % _NW == 0 and per_w % _C == 0 and B % _C == 0 and D % _L == 0
    assert chunks_per_f & (chunks_per_f - 1) == 0

    x_split = x.T.reshape(_NW, n_chunks, 1, _C)
    w_pad = jnp.zeros((_L,), jnp.float32).at[: weights.shape[0]].set(weights)

    mesh = plsc.VectorSubcoreMesh(core_axis_name="c", subcore_axis_name="s")

    @functools.partial(
        pl.kernel,
        mesh=mesh,
        out_type=jax.ShapeDtypeStruct((F, B, D), jnp.float32),
        scratch_types=[
            pltpu.VMEM((n_chunks, 1, _C), jnp.int32),
            pltpu.VMEM((_L,), jnp.float32),
            pltpu.VMEM((2, _C, D), jnp.float32),
            pltpu.SemaphoreType.DMA,
            pltpu.SemaphoreType.DMA,
            pltpu.SemaphoreType.DMA,
            pltpu.SemaphoreType.DMA,
        ],
    )
    def run(x_hbm, w_hbm, table_hbm, out_hbm, idx_v, w_v, rows_v, g0, g1, p0, p1):
        wid = lax.axis_index("s") * 2 + lax.axis_index("c")
        chunk0 = wid * n_chunks

        pltpu.sync_copy(w_hbm, w_v)
        pltpu.sync_copy(x_hbm.at[wid], idx_v)

        ones = jnp.ones((_L,), jnp.float32)
        w_vec = w_v[...]
        w0, w1, w2 = w_vec[0] * ones, w_vec[1] * ones, w_vec[2] * ones
        s2 = w2
        s1 = w1 + s2
        s0 = w0 + s1
        scales = [s0, s0, s1, s1, s2, s2, s2, s2]

        gsem = [g0, g1]
        wsem = [p0, p1]

        def gather(j):
            b = j % 2
            return pltpu.async_copy(table_hbm.at[idx_v.at[j, 0]], rows_v.at[b], gsem[b])

        def write(j):
            b = j % 2
            c = chunk0 + j
            f = c // chunks_per_f
            b0 = (c % chunks_per_f) * _C
            return pltpu.async_copy(
                rows_v.at[b], out_hbm.at[f, pl.ds(b0, _C)], wsem[b]
            )

        def scale_buf(b):
            @plsc.parallel_loop(0, _C, 1, unroll=4)
            def _(i):
                for jc in range(D // _L):
                    sl = pl.ds(jc * _L, _L)
                    rows_v[b, i, sl] = rows_v[b, i, sl] * scales[jc]

        pend_w = [None, None]
        gh = [None, None]
        gh[0] = gather(0)
        for j in range(n_chunks):
            b = j % 2
            nb = (j + 1) % 2
            if j + 1 < n_chunks:
                if pend_w[nb] is not None:
                    pend_w[nb].wait()
                    pend_w[nb] = None
                gh[nb] = gather(j + 1)
            gh[b].wait()
            scale_buf(b)
            pend_w[b] = write(j)
        for b in range(2):
            if pend_w[b] is not None:
                pend_w[b].wait()

    out_t = run(x_split, w_pad, table)
    return jnp.transpose(out_t, (1, 0, 2))


# speedup vs baseline: 2.8502x; 1.0985x over previous
import functools

import jax
import jax.numpy as jnp
from jax import lax
from jax.experimental import pallas as pl
from jax.experimental.pallas import tpu as pltpu
from jax.experimental.pallas import tpu_sc as plsc

_L = 16
_NW = 32
_C = 128


def kernel(x, weights, table):
    B, F = x.shape
    V, D = table.shape
    n_total = B * F
    per_w = n_total // _NW
    n_chunks = per_w // _C
    chunks_per_f = B // _C
    assert n_total % _NW == 0 and per_w % _C == 0 and B % _C == 0 and D % _L == 0
    assert chunks_per_f & (chunks_per_f - 1) == 0
    assert n_chunks % 2 == 0 and n_chunks >= 6

    x_split = x.T.reshape(_NW, n_chunks, 1, _C)
    w_pad = jnp.zeros((_L,), jnp.float32).at[: weights.shape[0]].set(weights)

    mesh = plsc.VectorSubcoreMesh(core_axis_name="c", subcore_axis_name="s")

    @functools.partial(
        pl.kernel,
        mesh=mesh,
        out_type=jax.ShapeDtypeStruct((F, B, D), jnp.float32),
        scratch_types=[
            pltpu.VMEM((n_chunks, 1, _C), jnp.int32),
            pltpu.VMEM((_L,), jnp.float32),
            pltpu.VMEM((2, _C, D), jnp.float32),
            pltpu.VMEM((2, _C, D), jnp.float32),
            pltpu.SemaphoreType.DMA,
            pltpu.SemaphoreType.DMA,
            pltpu.SemaphoreType.DMA,
            pltpu.SemaphoreType.DMA,
        ],
    )
    def run(x_hbm, w_hbm, table_hbm, out_hbm, idx_v, w_v, gbuf, obuf, g0, g1, p0, p1):
        wid = lax.axis_index("s") * 2 + lax.axis_index("c")
        chunk0 = wid * n_chunks

        pltpu.sync_copy(w_hbm, w_v)
        pltpu.sync_copy(x_hbm.at[wid], idx_v)

        ones = jnp.ones((_L,), jnp.float32)
        w_vec = w_v[...]
        w0, w1, w2 = w_vec[0] * ones, w_vec[1] * ones, w_vec[2] * ones
        s2 = w2
        s1 = w1 + s2
        s0 = w0 + s1
        scales = [s0, s0, s1, s1, s2, s2, s2, s2]

        gsem = [g0, g1]
        wsem = [p0, p1]

        def gather(j, p):
            return pltpu.async_copy(table_hbm.at[idx_v.at[j, 0]], gbuf.at[p], gsem[p])

        def write(j, p):
            c = chunk0 + j
            f = c // chunks_per_f
            b0 = (c % chunks_per_f) * _C
            return pltpu.async_copy(obuf.at[p], out_hbm.at[f, pl.ds(b0, _C)], wsem[p])

        def scale(p):
            @plsc.parallel_loop(0, _C, 1, unroll=4)
            def _(i):
                for jc in range(D // _L):
                    sl = pl.ds(jc * _L, _L)
                    obuf[p, i, sl] = gbuf[p, i, sl] * scales[jc]

        def wait_g(p):
            pltpu.make_async_copy(table_hbm.at[idx_v.at[0, 0]], gbuf.at[p], gsem[p]).wait()

        def wait_w(p):
            pltpu.make_async_copy(obuf.at[p], out_hbm.at[0, pl.ds(0, _C)], wsem[p]).wait()

        gather(0, 0)
        gather(1, 1)

        for p in range(2):
            wait_g(p)
            scale(p)
            gather(2 + p, p)
            write(p, p)

        def body(t, _):
            j = 2 * t
            for p in range(2):
                wait_g(p)
                wait_w(p)
                scale(p)
                gather(j + 2 + p, p)
                write(j + p, p)
            return 0

        lax.fori_loop(1, n_chunks // 2 - 1, body, 0)

        for p in range(2):
            jlast = n_chunks - 2 + p
            wait_g(p)
            wait_w(p)
            scale(p)
            write(jlast, p)
        for p in range(2):
            wait_w(p)

    out_t = run(x_split, w_pad, table)
    return jnp.transpose(out_t, (1, 0, 2))


# speedup vs baseline: 2.8729x; 1.0080x over previous
import functools

import jax
import jax.numpy as jnp
from jax import lax
from jax.experimental import pallas as pl
from jax.experimental.pallas import tpu as pltpu
from jax.experimental.pallas import tpu_sc as plsc

_L = 16
_NW = 32
_C = 128


def kernel(x, weights, table):
    B, F = x.shape
    V, D = table.shape
    n_total = B * F
    per_w = n_total // _NW
    n_chunks = per_w // _C
    chunks_per_f = B // _C
    assert n_total % _NW == 0 and per_w % _C == 0 and B % _C == 0 and D % _L == 0
    assert chunks_per_f & (chunks_per_f - 1) == 0
    assert n_chunks % 2 == 0 and n_chunks >= 6

    x_split = x.T.reshape(_NW, n_chunks, 1, _C)
    w_pad = jnp.zeros((_L,), jnp.float32).at[: weights.shape[0]].set(weights)

    mesh = plsc.VectorSubcoreMesh(core_axis_name="c", subcore_axis_name="s")

    @functools.partial(
        pl.kernel,
        mesh=mesh,
        out_type=jax.ShapeDtypeStruct((F, B, D), jnp.float32),
        scratch_types=[
            pltpu.VMEM((n_chunks, 1, _C), jnp.int32),
            pltpu.VMEM((_L,), jnp.float32),
            pltpu.VMEM((2, _C, D), jnp.float32),
            pltpu.VMEM((2, _C, D), jnp.float32),
            pltpu.SemaphoreType.DMA,
            pltpu.SemaphoreType.DMA,
            pltpu.SemaphoreType.DMA,
            pltpu.SemaphoreType.DMA,
        ],
    )
    def run(x_hbm, w_hbm, table_hbm, out_hbm, idx_v, w_v, gbuf, obuf, g0, g1, p0, p1):
        wid = lax.axis_index("s") * 2 + lax.axis_index("c")
        chunk0 = wid * n_chunks

        pltpu.sync_copy(w_hbm, w_v)
        pltpu.sync_copy(x_hbm.at[wid], idx_v)

        ones = jnp.ones((_L,), jnp.float32)
        w_vec = w_v[...]
        w0, w1, w2 = w_vec[0] * ones, w_vec[1] * ones, w_vec[2] * ones
        s2 = w2
        s1 = w1 + s2
        s0 = w0 + s1
        scales = [s0, s0, s1, s1, s2, s2, s2, s2]

        gsem = [g0, g1]
        wsem = [p0, p1]

        def gather(j, p):
            return pltpu.async_copy(table_hbm.at[idx_v.at[j, 0]], gbuf.at[p], gsem[p])

        def write(j, p):
            c = chunk0 + j
            f = c // chunks_per_f
            b0 = (c % chunks_per_f) * _C
            return pltpu.async_copy(obuf.at[p], out_hbm.at[f, pl.ds(b0, _C)], wsem[p])

        def scale(p):
            @plsc.parallel_loop(0, _C, 1, unroll=4)
            def _(i):
                for jc in range(D // _L):
                    sl = pl.ds(jc * _L, _L)
                    obuf[p, i, sl] = gbuf[p, i, sl] * scales[jc]

        def wait_g(p):
            pltpu.make_async_copy(table_hbm.at[idx_v.at[0, 0]], gbuf.at[p], gsem[p]).wait()

        def wait_w(p):
            pltpu.make_async_copy(obuf.at[p], out_hbm.at[0, pl.ds(0, _C)], wsem[p]).wait()

        gather(0, 0)
        gather(1, 1)

        def body(t, _):
            j = 2 * t
            for p in range(2):
                wait_g(p)

                @pl.when(t > 0)
                def _():
                    wait_w(p)

                scale(p)

                @pl.when(j + 2 + p < n_chunks)
                def _():
                    gather(j + 2 + p, p)

                write(j + p, p)
            return 0

        lax.fori_loop(0, n_chunks // 2, body, 0)
        for p in range(2):
            wait_w(p)

    out_t = run(x_split, w_pad, table)
    return jnp.transpose(out_t, (1, 0, 2))


# speedup vs baseline: 2.9101x; 1.0130x over previous
import functools

import jax
import jax.numpy as jnp
from jax import lax
from jax.experimental import pallas as pl
from jax.experimental.pallas import tpu as pltpu
from jax.experimental.pallas import tpu_sc as plsc

_L = 16
_NW = 32
_C = 128


def kernel(x, weights, table):
    B, F = x.shape
    V, D = table.shape
    n_total = B * F
    per_w = n_total // _NW
    n_chunks = per_w // _C
    chunks_per_f = B // _C
    assert n_total % _NW == 0 and per_w % _C == 0 and B % _C == 0 and D % _L == 0
    assert chunks_per_f & (chunks_per_f - 1) == 0
    assert n_chunks % 2 == 0 and n_chunks >= 6

    x_split = x.T.reshape(_NW, n_chunks, 1, _C)
    w_pad = jnp.zeros((_L,), jnp.float32).at[: weights.shape[0]].set(weights)

    mesh = plsc.VectorSubcoreMesh(core_axis_name="c", subcore_axis_name="s")

    @functools.partial(
        pl.kernel,
        mesh=mesh,
        out_type=jax.ShapeDtypeStruct((F, B, D), jnp.float32),
        scratch_types=[
            pltpu.VMEM((n_chunks, 1, _C), jnp.int32),
            pltpu.VMEM((_L,), jnp.float32),
            pltpu.VMEM((4, _C, D), jnp.float32),
            pltpu.VMEM((2, _C, D), jnp.float32),
            pltpu.SemaphoreType.DMA,
            pltpu.SemaphoreType.DMA,
            pltpu.SemaphoreType.DMA,
            pltpu.SemaphoreType.DMA,
            pltpu.SemaphoreType.DMA,
            pltpu.SemaphoreType.DMA,
        ],
    )
    def run(x_hbm, w_hbm, table_hbm, out_hbm, idx_v, w_v, gbuf, obuf, g0, g1, g2, g3, p0, p1):
        wid = lax.axis_index("s") * 2 + lax.axis_index("c")
        chunk0 = wid * n_chunks

        pltpu.sync_copy(w_hbm, w_v)
        pltpu.sync_copy(x_hbm.at[wid], idx_v)

        ones = jnp.ones((_L,), jnp.float32)
        w_vec = w_v[...]
        w0, w1, w2 = w_vec[0] * ones, w_vec[1] * ones, w_vec[2] * ones
        s2 = w2
        s1 = w1 + s2
        s0 = w0 + s1
        scales = [s0, s0, s1, s1, s2, s2, s2, s2]

        gsem = [g0, g1, g2, g3]
        wsem = [p0, p1]

        def gather(j, p):
            return pltpu.async_copy(table_hbm.at[idx_v.at[j, 0]], gbuf.at[p], gsem[p])

        def write(j, op):
            c = chunk0 + j
            f = c // chunks_per_f
            b0 = (c % chunks_per_f) * _C
            return pltpu.async_copy(obuf.at[op], out_hbm.at[f, pl.ds(b0, _C)], wsem[op])

        def scale(p, op):
            @plsc.parallel_loop(0, _C, 1, unroll=4)
            def _(i):
                for jc in range(D // _L):
                    sl = pl.ds(jc * _L, _L)
                    obuf[op, i, sl] = gbuf[p, i, sl] * scales[jc]

        def wait_g(p):
            pltpu.make_async_copy(table_hbm.at[idx_v.at[0, 0]], gbuf.at[p], gsem[p]).wait()

        def wait_w(op):
            pltpu.make_async_copy(obuf.at[op], out_hbm.at[0, pl.ds(0, _C)], wsem[op]).wait()

        for p in range(4):
            gather(p, p)

        def body(t, _):
            j = 4 * t
            for p in range(4):
                jq = j + p
                op = p % 2

                @pl.when(jq < n_chunks)
                def _():
                    wait_g(p)

                    @pl.when(jq >= 2)
                    def _():
                        wait_w(op)

                    scale(p, op)

                    @pl.when(jq + 4 < n_chunks)
                    def _():
                        gather(jq + 4, p)

                    write(jq, op)

            return 0

        lax.fori_loop(0, (n_chunks + 3) // 4, body, 0)
        for op in range(2):
            wait_w(op)

    out_t = run(x_split, w_pad, table)
    return jnp.transpose(out_t, (1, 0, 2))


# speedup vs baseline: 2.9170x; 1.0024x over previous
import functools

import jax
import jax.numpy as jnp
from jax import lax
from jax.experimental import pallas as pl
from jax.experimental.pallas import tpu as pltpu
from jax.experimental.pallas import tpu_sc as plsc

_L = 16
_NW = 32
_C = 128


def kernel(x, weights, table):
    B, F = x.shape
    V, D = table.shape
    n_total = B * F
    per_w = n_total // _NW
    n_chunks = per_w // _C
    chunks_per_f = B // _C
    assert n_total % _NW == 0 and per_w % _C == 0 and B % _C == 0 and D % _L == 0
    assert chunks_per_f & (chunks_per_f - 1) == 0
    assert n_chunks % 2 == 0 and n_chunks >= 6

    x_split = x.T.reshape(_NW, n_chunks, 1, _C)
    w_pad = jnp.zeros((_L,), jnp.float32).at[: weights.shape[0]].set(weights)

    mesh = plsc.VectorSubcoreMesh(core_axis_name="c", subcore_axis_name="s")

    @functools.partial(
        pl.kernel,
        mesh=mesh,
        out_type=jax.ShapeDtypeStruct((F, B, D), jnp.float32),
        scratch_types=[
            pltpu.VMEM((n_chunks, 1, _C), jnp.int32),
            pltpu.VMEM((_L,), jnp.float32),
            pltpu.VMEM((3, _C, D), jnp.float32),
            pltpu.VMEM((3, _C, D), jnp.float32),
            pltpu.SemaphoreType.DMA,
            pltpu.SemaphoreType.DMA,
            pltpu.SemaphoreType.DMA,
            pltpu.SemaphoreType.DMA,
            pltpu.SemaphoreType.DMA,
            pltpu.SemaphoreType.DMA,
        ],
    )
    def run(x_hbm, w_hbm, table_hbm, out_hbm, idx_v, w_v, gbuf, obuf, g0, g1, g2, g3, p0, p1):
        wid = lax.axis_index("s") * 2 + lax.axis_index("c")
        chunk0 = wid * n_chunks

        pltpu.sync_copy(w_hbm, w_v)
        pltpu.sync_copy(x_hbm.at[wid], idx_v)

        ones = jnp.ones((_L,), jnp.float32)
        w_vec = w_v[...]
        w0, w1, w2 = w_vec[0] * ones, w_vec[1] * ones, w_vec[2] * ones
        s2 = w2
        s1 = w1 + s2
        s0 = w0 + s1
        scales = [s0, s0, s1, s1, s2, s2, s2, s2]

        gsem = [g0, g1, g2]
        wsem = [p0, p1, g3]

        def gather(j, p):
            return pltpu.async_copy(table_hbm.at[idx_v.at[j, 0]], gbuf.at[p], gsem[p])

        def write(j, op):
            c = chunk0 + j
            f = c // chunks_per_f
            b0 = (c % chunks_per_f) * _C
            return pltpu.async_copy(obuf.at[op], out_hbm.at[f, pl.ds(b0, _C)], wsem[op])

        def scale(p, op):
            @plsc.parallel_loop(0, _C, 1, unroll=4)
            def _(i):
                for jc in range(D // _L):
                    sl = pl.ds(jc * _L, _L)
                    obuf[op, i, sl] = gbuf[p, i, sl] * scales[jc]

        def wait_g(p):
            pltpu.make_async_copy(table_hbm.at[idx_v.at[0, 0]], gbuf.at[p], gsem[p]).wait()

        def wait_w(op):
            pltpu.make_async_copy(obuf.at[op], out_hbm.at[0, pl.ds(0, _C)], wsem[op]).wait()

        for p in range(3):
            gather(p, p)

        def body(t, _):
            j = 3 * t
            for p in range(3):
                jq = j + p

                @pl.when(jq < n_chunks)
                def _():
                    wait_g(p)

                    @pl.when(jq >= 3)
                    def _():
                        wait_w(p)

                    scale(p, p)

                    @pl.when(jq + 3 < n_chunks)
                    def _():
                        gather(jq + 3, p)

                    write(jq, p)

            return 0

        lax.fori_loop(0, (n_chunks + 2) // 3, body, 0)
        for op in range(3):
            wait_w(op)

    out_t = run(x_split, w_pad, table)
    return jnp.transpose(out_t, (1, 0, 2))
